# baseline (speedup 1.0000x reference)
import jax
import jax.numpy as jnp
from jax.experimental import pallas as pl
from jax.experimental.pallas import tpu as pltpu

_BM = 400


def _layer1_body(a_ref, x_ref, w1_ref, w2_ref, g_ref, q_ref, cs_ref, p_ref):
    i = pl.program_id(0)

    @pl.when(i == 0)
    def _compute_p():
        p_ref[...] = jnp.dot(x_ref[...], w1_ref[...],
                             preferred_element_type=jnp.float32)

    a = a_ref[...]
    h = jnp.dot(a, p_ref[...], preferred_element_type=jnp.float32)
    h = jnp.maximum(h, 0.0)
    g = jnp.dot(h, w2_ref[...], preferred_element_type=jnp.float32)
    g_ref[...] = g.astype(jnp.bfloat16)
    q_ref[0] = jnp.round(a * 254.0 - 127.0).astype(jnp.int8)
    blk_cs = jnp.sum(g, axis=0, keepdims=True)

    @pl.when(i == 0)
    def _init():
        cs_ref[...] = blk_cs

    @pl.when(i != 0)
    def _acc():
        cs_ref[...] += blk_cs


def _layer2_body(q_ref, g_ref, cs_ref, w_ref, b_ref, o_ref):
    g = g_ref[...]
    for s in range(q_ref.shape[0]):
        qb = q_ref[s].astype(jnp.bfloat16)
        acc = jnp.dot(qb, g, preferred_element_type=jnp.float32)
        h = (acc + 127.0 * cs_ref[...]) * (1.0 / 254.0)
        h = jnp.maximum(h, 0.0)
        o_ref[s * _BM:(s + 1) * _BM, :] = (
            jnp.dot(h, w_ref[...], preferred_element_type=jnp.float32)
            + b_ref[0, 0])


def kernel(adj, x, W1, W2, W3, b3):
    n, d_in = x.shape
    h1 = W1.shape[1]
    h2 = W2.shape[1]
    nblk = n // _BM

    grid = (nblk,)
    g, q, cs = pl.pallas_call(
        _layer1_body,
        grid=grid,
        in_specs=[
            pl.BlockSpec((_BM, n), lambda i: (i, 0)),
            pl.BlockSpec((n, d_in), lambda i: (0, 0)),
            pl.BlockSpec((d_in, h1), lambda i: (0, 0)),
            pl.BlockSpec((h1, h2), lambda i: (0, 0)),
        ],
        out_specs=[
            pl.BlockSpec((_BM, h2), lambda i: (i, 0)),
            pl.BlockSpec((1, _BM, n), lambda i: (i, 0, 0)),
            pl.BlockSpec((1, h2), lambda i: (0, 0)),
        ],
        out_shape=[
            jax.ShapeDtypeStruct((n, h2), jnp.bfloat16),
            jax.ShapeDtypeStruct((nblk, _BM, n), jnp.int8),
            jax.ShapeDtypeStruct((1, h2), jnp.float32),
        ],
        scratch_shapes=[pltpu.VMEM((n, h1), jnp.float32)],
    )(adj, x, W1, W2)

    nsub = 5 if nblk % 5 == 0 else 1
    out = pl.pallas_call(
        _layer2_body,
        grid=(nblk // nsub,),
        in_specs=[
            pl.BlockSpec((nsub, _BM, n), lambda i: (i, 0, 0)),
            pl.BlockSpec((n, h2), lambda i: (0, 0)),
            pl.BlockSpec((1, h2), lambda i: (0, 0)),
            pl.BlockSpec((h2, 1), lambda i: (0, 0)),
            pl.BlockSpec((1, 1), lambda i: (0, 0)),
        ],
        out_specs=pl.BlockSpec((nsub * _BM, 1), lambda i: (i, 0)),
        out_shape=jax.ShapeDtypeStruct((n, 1), jnp.float32),
    )(q, g, cs, W3, b3.reshape(1, 1))
    return out


# speedup vs baseline: 1.1674x; 1.1674x over previous
import jax
import jax.numpy as jnp
from jax.experimental import pallas as pl
from jax.experimental.pallas import tpu as pltpu

_BM = 200


def _layer1_body(a_ref, x_ref, w1_ref, w2_ref, g_ref, q_ref, cs_ref, p_ref):
    i = pl.program_id(0)

    @pl.when(i == 0)
    def _compute_p():
        p_ref[...] = jnp.dot(x_ref[...], w1_ref[...],
                             preferred_element_type=jnp.float32)

    a = a_ref[...]
    h = jnp.dot(a, p_ref[...], preferred_element_type=jnp.float32)
    h = jnp.maximum(h, 0.0)
    g = jnp.dot(h, w2_ref[...], preferred_element_type=jnp.float32)
    g_ref[...] = g.astype(jnp.bfloat16)
    q_ref[...] = jnp.round(a * 254.0 - 127.0).astype(jnp.int8)
    blk_cs = jnp.sum(g, axis=0, keepdims=True)

    @pl.when(i == 0)
    def _init():
        cs_ref[...] = blk_cs

    @pl.when(i != 0)
    def _acc():
        cs_ref[...] += blk_cs


def _layer2_body(q_ref, g_ref, cs_ref, w_ref, b_ref, o_ref):
    qb = q_ref[...].astype(jnp.bfloat16)
    s = jnp.dot(qb, g_ref[...], preferred_element_type=jnp.float32)
    h = (s + 127.0 * cs_ref[...]) * (1.0 / 254.0)
    h = jnp.maximum(h, 0.0)
    o_ref[...] = (jnp.dot(h, w_ref[...], preferred_element_type=jnp.float32)
                  + b_ref[0, 0])


def kernel(adj, x, W1, W2, W3, b3):
    n, d_in = x.shape
    h1 = W1.shape[1]
    h2 = W2.shape[1]
    nblk = n // _BM

    grid = (nblk,)
    g, q, cs = pl.pallas_call(
        _layer1_body,
        grid=grid,
        in_specs=[
            pl.BlockSpec((_BM, n), lambda i: (i, 0)),
            pl.BlockSpec((n, d_in), lambda i: (0, 0)),
            pl.BlockSpec((d_in, h1), lambda i: (0, 0)),
            pl.BlockSpec((h1, h2), lambda i: (0, 0)),
        ],
        out_specs=[
            pl.BlockSpec((_BM, h2), lambda i: (i, 0)),
            pl.BlockSpec((_BM, n), lambda i: (i, 0)),
            pl.BlockSpec((1, h2), lambda i: (0, 0)),
        ],
        out_shape=[
            jax.ShapeDtypeStruct((n, h2), jnp.bfloat16),
            jax.ShapeDtypeStruct((n, n), jnp.int8),
            jax.ShapeDtypeStruct((1, h2), jnp.float32),
        ],
        scratch_shapes=[pltpu.VMEM((n, h1), jnp.float32)],
    )(adj, x, W1, W2)

    nsub = 5 if nblk % 5 == 0 else 1
    out = pl.pallas_call(
        _layer2_body,
        grid=(nblk // nsub,),
        in_specs=[
            pl.BlockSpec((nsub * _BM, n), lambda i: (i, 0)),
            pl.BlockSpec((n, h2), lambda i: (0, 0)),
            pl.BlockSpec((1, h2), lambda i: (0, 0)),
            pl.BlockSpec((h2, 1), lambda i: (0, 0)),
            pl.BlockSpec((1, 1), lambda i: (0, 0)),
        ],
        out_specs=pl.BlockSpec((nsub * _BM, 1), lambda i: (i, 0)),
        out_shape=jax.ShapeDtypeStruct((n, 1), jnp.float32),
    )(q, g, cs, W3, b3.reshape(1, 1))
    return out


# speedup vs baseline: 1.1772x; 1.0084x over previous
import jax
import jax.numpy as jnp
from jax.experimental import pallas as pl
from jax.experimental.pallas import tpu as pltpu

_BM = 400


def _layer1_body(a_ref, x_ref, w1_ref, w2_ref, g_ref, q_ref, cs_ref, p_ref):
    i = pl.program_id(0)

    @pl.when(i == 0)
    def _compute_p():
        p_ref[...] = jnp.dot(x_ref[...], w1_ref[...],
                             preferred_element_type=jnp.float32)

    a = a_ref[...]
    h = jnp.dot(a, p_ref[...], preferred_element_type=jnp.float32)
    h = jnp.maximum(h, 0.0)
    g = jnp.dot(h, w2_ref[...], preferred_element_type=jnp.float32)
    g_ref[...] = g.astype(jnp.bfloat16)
    q_ref[0] = jnp.round(a * 254.0 - 127.0).astype(jnp.int8)
    blk_cs = jnp.sum(g, axis=0, keepdims=True)

    @pl.when(i == 0)
    def _init():
        cs_ref[...] = blk_cs

    @pl.when(i != 0)
    def _acc():
        cs_ref[...] += blk_cs


def _layer2_body(q_ref, g_ref, cs_ref, w_ref, b_ref, o_ref):
    qb = q_ref[...].astype(jnp.bfloat16)
    s = jnp.dot(qb, g_ref[...], preferred_element_type=jnp.float32)
    h = (s + 127.0 * cs_ref[...]) * (1.0 / 254.0)
    h = jnp.maximum(h, 0.0)
    o_ref[...] = (jnp.dot(h, w_ref[...], preferred_element_type=jnp.float32)
                  + b_ref[0, 0])


def kernel(adj, x, W1, W2, W3, b3):
    n, d_in = x.shape
    h1 = W1.shape[1]
    h2 = W2.shape[1]
    nblk = n // _BM

    grid = (nblk,)
    g, q, cs = pl.pallas_call(
        _layer1_body,
        grid=grid,
        in_specs=[
            pl.BlockSpec((_BM, n), lambda i: (i, 0)),
            pl.BlockSpec((n, d_in), lambda i: (0, 0)),
            pl.BlockSpec((d_in, h1), lambda i: (0, 0)),
            pl.BlockSpec((h1, h2), lambda i: (0, 0)),
        ],
        out_specs=[
            pl.BlockSpec((_BM, h2), lambda i: (i, 0)),
            pl.BlockSpec((1, _BM, n), lambda i: (i, 0, 0)),
            pl.BlockSpec((1, h2), lambda i: (0, 0)),
        ],
        out_shape=[
            jax.ShapeDtypeStruct((n, h2), jnp.bfloat16),
            jax.ShapeDtypeStruct((nblk, _BM, n), jnp.int8),
            jax.ShapeDtypeStruct((1, h2), jnp.float32),
        ],
        scratch_shapes=[pltpu.VMEM((n, h1), jnp.float32)],
    )(adj, x, W1, W2)

    q = q.reshape(n, n)
    nsub = 5 if nblk % 5 == 0 else 1
    out = pl.pallas_call(
        _layer2_body,
        grid=(nblk // nsub,),
        in_specs=[
            pl.BlockSpec((nsub * _BM, n), lambda i: (i, 0)),
            pl.BlockSpec((n, h2), lambda i: (0, 0)),
            pl.BlockSpec((1, h2), lambda i: (0, 0)),
            pl.BlockSpec((h2, 1), lambda i: (0, 0)),
            pl.BlockSpec((1, 1), lambda i: (0, 0)),
        ],
        out_specs=pl.BlockSpec((nsub * _BM, 1), lambda i: (i, 0)),
        out_shape=jax.ShapeDtypeStruct((n, 1), jnp.float32),
    )(q, g, cs, W3, b3.reshape(1, 1))
    return out
